# baseline (speedup 1.0000x reference)
import functools
import math

import jax
import jax.numpy as jnp
import numpy as np
from jax.experimental import pallas as pl
from jax.experimental.pallas import tpu as pltpu

ACTUAL_DIM = 768
PADDED_DIM = 1024
GROUP = 256
NUM_LEVELS = 16
SIGMA = 1.0
_HALF = (NUM_LEVELS - 1) / 2.0


def _hadamard(n):
    H = np.array([[1.0]], dtype=np.float32)
    while H.shape[0] < n:
        H = np.block([[H, H], [H, -H]]).astype(np.float32)
    return H / np.sqrt(np.float32(n))


_H = _hadamard(PADDED_DIM)
_SIGNS = np.random.RandomState(1234).choice(
    np.array([-1.0, 1.0], dtype=np.float32), size=(PADDED_DIM,)
).astype(np.float32)

_H256 = (_hadamard(GROUP) * (np.sqrt(np.float32(GROUP)) / 32.0)).astype(np.float32)

_HF = np.stack([
    _SIGNS[a * GROUP:(a + 1) * GROUP][:, None] * _H256 for a in range(3)
], axis=0).astype(np.float32)

_SOUT = _SIGNS[:ACTUAL_DIM].reshape(3, 1, GROUP).astype(np.float32)

_CORR = np.zeros((1, GROUP), dtype=np.int32)
_CORR[0, 0] = int(7.5 * 4.0 * GROUP)

_H256U = np.rint(_hadamard(GROUP) * np.sqrt(np.float32(GROUP))).astype(np.int8)

_SOUT32 = (_SOUT / 32.0).astype(np.float32)


_CHUNK = 2048


def _fused_kernel(x_ref, w_ref, b_ref, hf_ref, hm_ref, s_ref, corr_ref,
                  out_ref, hc_ref):
    @pl.when(pl.program_id(0) == 0)
    def _bias_rows():
        pb = [
            jnp.dot(b_ref[...][:, a * GROUP:(a + 1) * GROUP], hf_ref[a],
                    preferred_element_type=jnp.float32)
            for a in range(3)
        ]
        c0 = pb[0] + pb[1]
        c1 = pb[0] - pb[1]
        hc_ref[0] = c0 + pb[2] + _HALF
        hc_ref[1] = c1 + pb[2] + _HALF
        hc_ref[2] = c0 - pb[2] + _HALF
        hc_ref[3] = c1 - pb[2] + _HALF

    y = jnp.dot(x_ref[...], w_ref[...], preferred_element_type=jnp.float32)
    hc = [hc_ref[0], hc_ref[1], hc_ref[2], hc_ref[3]]

    n_chunks = y.shape[0] // _CHUNK
    for ci in range(n_chunks):
        rows = pl.ds(ci * _CHUNK, _CHUNK)

        p = [
            jnp.dot(y[ci * _CHUNK:(ci + 1) * _CHUNK,
                      a * GROUP:(a + 1) * GROUP], hf_ref[a],
                    preferred_element_type=jnp.float32)
            for a in range(3)
        ]
        a0 = p[0] + p[1]
        a1 = p[0] - p[1]
        r = [a0 + p[2], a1 + p[2], a0 - p[2], a1 - p[2]]

        q = [
            jnp.clip(jnp.round(rg + hcg), 0.0, NUM_LEVELS - 1.0)
            .astype(jnp.int8)
            for rg, hcg in zip(r, hc)
        ]

        t = [
            jnp.dot(qg, hm_ref[...], preferred_element_type=jnp.int32)
            for qg in q
        ]
        b0 = t[0] + t[1]
        b1 = t[0] - t[1]
        b2 = t[2] + t[3]
        b3 = t[2] - t[3]
        out_ref[rows, 0:GROUP] = (
            (b0 + b2 - corr_ref[...]).astype(jnp.float32) * s_ref[0])
        out_ref[rows, GROUP:2 * GROUP] = (
            (b1 + b3).astype(jnp.float32) * s_ref[1])
        out_ref[rows, 2 * GROUP:3 * GROUP] = (
            (b0 - b2).astype(jnp.float32) * s_ref[2])


@functools.partial(jax.jit, static_argnames=("block_m",))
def _run(x2d, W, b2d, hf, hm, souts, corr, block_m):
    n_tok = x2d.shape[0]
    grid = (n_tok // block_m,)
    return pl.pallas_call(
        _fused_kernel,
        grid=grid,
        in_specs=[
            pl.BlockSpec((block_m, ACTUAL_DIM), lambda i: (i, 0)),
            pl.BlockSpec((ACTUAL_DIM, ACTUAL_DIM), lambda i: (0, 0)),
            pl.BlockSpec((1, ACTUAL_DIM), lambda i: (0, 0)),
            pl.BlockSpec((3, GROUP, GROUP), lambda i: (0, 0, 0)),
            pl.BlockSpec((GROUP, GROUP), lambda i: (0, 0)),
            pl.BlockSpec((3, 1, GROUP), lambda i: (0, 0, 0)),
            pl.BlockSpec((1, GROUP), lambda i: (0, 0)),
        ],
        out_specs=pl.BlockSpec((block_m, ACTUAL_DIM), lambda i: (i, 0)),
        out_shape=jax.ShapeDtypeStruct((n_tok, ACTUAL_DIM), jnp.float32),
        scratch_shapes=[pltpu.VMEM((4, 1, GROUP), jnp.float32)],
    )(x2d, W, b2d, hf, hm, souts, corr)


def kernel(x, W, b):
    batch, seq, dim = x.shape
    x2d = x.reshape(batch * seq, dim)
    b2d = b.reshape(1, dim)
    wbf = W.astype(jnp.bfloat16)
    hf = jnp.asarray(_HF).astype(jnp.bfloat16)
    hm = jnp.asarray(_H256U)
    souts = jnp.asarray(_SOUT32)
    corr = jnp.asarray(_CORR)
    out = _run(x2d, wbf, b2d, hf, hm, souts, corr, 2048)
    return out.reshape(batch, seq, dim)


# speedup vs baseline: 1.1521x; 1.1521x over previous
import functools
import math

import jax
import jax.numpy as jnp
import numpy as np
from jax.experimental import pallas as pl
from jax.experimental.pallas import tpu as pltpu

ACTUAL_DIM = 768
PADDED_DIM = 1024
GROUP = 256
NUM_LEVELS = 16
SIGMA = 1.0
_HALF = (NUM_LEVELS - 1) / 2.0


def _hadamard(n):
    H = np.array([[1.0]], dtype=np.float32)
    while H.shape[0] < n:
        H = np.block([[H, H], [H, -H]]).astype(np.float32)
    return H / np.sqrt(np.float32(n))


_H = _hadamard(PADDED_DIM)
_SIGNS = np.random.RandomState(1234).choice(
    np.array([-1.0, 1.0], dtype=np.float32), size=(PADDED_DIM,)
).astype(np.float32)

_H256 = (_hadamard(GROUP) * (np.sqrt(np.float32(GROUP)) / 32.0)).astype(np.float32)

_HF = np.stack([
    _SIGNS[a * GROUP:(a + 1) * GROUP][:, None] * _H256 for a in range(3)
], axis=0).astype(np.float32)

_SOUT = _SIGNS[:ACTUAL_DIM].reshape(3, 1, GROUP).astype(np.float32)

_CORR = np.zeros((1, GROUP), dtype=np.float32)
_CORR[0, 0] = 7.5 * 4.0 * (GROUP / 32.0)


_CHUNK = 2048


def _fused_kernel(x_ref, w_ref, b_ref, hf_ref, hm_ref, s_ref, corr_ref,
                  out_ref, hc_ref):
    @pl.when(pl.program_id(0) == 0)
    def _bias_rows():
        pb = [
            jnp.dot(b_ref[...][:, a * GROUP:(a + 1) * GROUP], hf_ref[a],
                    preferred_element_type=jnp.float32)
            for a in range(3)
        ]
        c0 = pb[0] + pb[1]
        c1 = pb[0] - pb[1]
        hc_ref[0] = c0 + pb[2] + _HALF
        hc_ref[1] = c1 + pb[2] + _HALF
        hc_ref[2] = c0 - pb[2] + _HALF
        hc_ref[3] = c1 - pb[2] + _HALF

    y = jnp.dot(x_ref[...], w_ref[...], preferred_element_type=jnp.float32)
    hc = [hc_ref[0], hc_ref[1], hc_ref[2], hc_ref[3]]

    n_chunks = y.shape[0] // _CHUNK
    for ci in range(n_chunks):
        rows = pl.ds(ci * _CHUNK, _CHUNK)

        p = [
            jnp.dot(y[ci * _CHUNK:(ci + 1) * _CHUNK,
                      a * GROUP:(a + 1) * GROUP], hf_ref[a],
                    preferred_element_type=jnp.float32)
            for a in range(3)
        ]
        a0 = p[0] + p[1]
        a1 = p[0] - p[1]
        r = [a0 + p[2], a1 + p[2], a0 - p[2], a1 - p[2]]

        q = [
            jnp.clip(jnp.round(rg + hcg), 0.0, NUM_LEVELS - 1.0)
            .astype(jnp.bfloat16)
            for rg, hcg in zip(r, hc)
        ]

        t = [
            jnp.dot(qg, hm_ref[...], preferred_element_type=jnp.float32)
            for qg in q
        ]
        b0 = t[0] + t[1]
        b1 = t[0] - t[1]
        b2 = t[2] + t[3]
        b3 = t[2] - t[3]
        out_ref[rows, 0:GROUP] = (b0 + b2 - corr_ref[...]) * s_ref[0]
        out_ref[rows, GROUP:2 * GROUP] = (b1 + b3) * s_ref[1]
        out_ref[rows, 2 * GROUP:3 * GROUP] = (b0 - b2) * s_ref[2]


@functools.partial(jax.jit, static_argnames=("block_m",))
def _run(x2d, W, b2d, hf, hm, souts, corr, block_m):
    n_tok = x2d.shape[0]
    grid = (n_tok // block_m,)
    return pl.pallas_call(
        _fused_kernel,
        grid=grid,
        in_specs=[
            pl.BlockSpec((block_m, ACTUAL_DIM), lambda i: (i, 0)),
            pl.BlockSpec((ACTUAL_DIM, ACTUAL_DIM), lambda i: (0, 0)),
            pl.BlockSpec((1, ACTUAL_DIM), lambda i: (0, 0)),
            pl.BlockSpec((3, GROUP, GROUP), lambda i: (0, 0, 0)),
            pl.BlockSpec((GROUP, GROUP), lambda i: (0, 0)),
            pl.BlockSpec((3, 1, GROUP), lambda i: (0, 0, 0)),
            pl.BlockSpec((1, GROUP), lambda i: (0, 0)),
        ],
        out_specs=pl.BlockSpec((block_m, ACTUAL_DIM), lambda i: (i, 0)),
        out_shape=jax.ShapeDtypeStruct((n_tok, ACTUAL_DIM), jnp.float32),
        scratch_shapes=[pltpu.VMEM((4, 1, GROUP), jnp.float32)],
    )(x2d, W, b2d, hf, hm, souts, corr)


def kernel(x, W, b):
    batch, seq, dim = x.shape
    x2d = x.reshape(batch * seq, dim)
    b2d = b.reshape(1, dim)
    wbf = W.astype(jnp.bfloat16)
    hf = jnp.asarray(_HF).astype(jnp.bfloat16)
    hm = jnp.asarray(_H256).astype(jnp.bfloat16)
    souts = jnp.asarray(_SOUT)
    corr = jnp.asarray(_CORR)
    out = _run(x2d, wbf, b2d, hf, hm, souts, corr, 2048)
    return out.reshape(batch, seq, dim)
